# Initial kernel scaffold; baseline (speedup 1.0000x reference)
import jax
import jax.numpy as jnp
from jax.experimental import pallas as pl


def kernel(x, edge_index, W_l0, b_l0, W_r0, b_r0, gamma0, beta0, W_l1, b_l1, W_r1, b_r1, gamma1, beta1, W_l2, b_l2, W_r2, b_r2, gamma2, beta2):
    raise NotImplementedError("write your pallas kernel here")


# speedup vs baseline: 2.9290x; 2.9290x over previous
import jax
import jax.numpy as jnp
from jax import lax
from jax.experimental import pallas as pl
from jax.experimental.pallas import tpu as pltpu
from jax.experimental.pallas import tpu_sc as plsc

N = 10000
E = 320000
D_IN = 128
H = 256

N_PAD = 10240
E_PAD = 327680
IDX_ROWS = E_PAD // 128
STRIPE = N_PAD // 16
BN = 400
GRID = N // BN

import functools


@functools.lru_cache(maxsize=None)
def _mesh():
    return plsc.VectorSubcoreMesh(core_axis_name="c", subcore_axis_name="s")


def _agg_edge_body(table, srcb, dstb, zrow, out,
                   acc, src_i, dst_i, rows, sem):
    c = lax.axis_index("c")
    s = lax.axis_index("s")
    w = c * 16 + s
    pltpu.sync_copy(zrow, rows)
    for k in range(STRIPE // 128):
        pltpu.sync_copy(rows, acc.at[pl.ds(s * STRIPE + k * 128, 128)])
    plsc.subcore_barrier()

    rpw = IDX_ROWS // 32

    def step(g, carry):
        r0 = w * rpw + g * 8
        pltpu.sync_copy(srcb.at[pl.ds(r0, 8)], src_i)
        pltpu.sync_copy(dstb.at[pl.ds(r0, 8)], dst_i)
        for j in range(8):
            pltpu.async_copy(table.at[src_i.at[j]], rows, sem).wait()
            pltpu.sync_copy(rows, acc.at[dst_i.at[j]], add=True)
        return carry

    lax.fori_loop(0, rpw // 8, step, 0)
    plsc.subcore_barrier()
    for k in range(STRIPE // 128):
        off = s * STRIPE + k * 128
        pltpu.sync_copy(acc.at[pl.ds(off, 128)], rows)
        pltpu.sync_copy(rows, out.at[pl.ds(c * N_PAD + off, 128)])


@functools.lru_cache(maxsize=None)
def _agg0():
    return pl.kernel(
    _agg_edge_body,
    out_type=jax.ShapeDtypeStruct((2 * N_PAD, 128), jnp.float32),
    mesh=_mesh(),
    scratch_types=[
        pltpu.VMEM_SHARED((N_PAD, 128), jnp.float32),
        pltpu.VMEM((8, 128), jnp.int32),
        pltpu.VMEM((8, 128), jnp.int32),
        pltpu.VMEM((128, 128), jnp.float32),
        pltpu.SemaphoreType.DMA,
    ],
    )


def _deg_body(dstb, zrow, ones128, out, dacc, dst_i, buf, sem):
    c = lax.axis_index("c")
    s = lax.axis_index("s")
    w = c * 16 + s
    pltpu.sync_copy(zrow, buf)
    for k in range(STRIPE // 128):
        pltpu.sync_copy(buf, dacc.at[pl.ds(s * STRIPE + k * 128, 128)])
    pltpu.sync_copy(ones128, buf)
    plsc.subcore_barrier()

    rpw = IDX_ROWS // 32

    def step(g, carry):
        pltpu.sync_copy(dstb.at[pl.ds(w * rpw + g * 8, 8)], dst_i)
        for j in range(8):
            pltpu.sync_copy(buf, dacc.at[dst_i.at[j]], add=True)
        return carry

    lax.fori_loop(0, rpw // 8, step, 0)
    plsc.subcore_barrier()
    for k in range(STRIPE // 128):
        off = s * STRIPE + k * 128
        pltpu.sync_copy(dacc.at[pl.ds(off, 128)], buf)
        pltpu.sync_copy(buf, out.at[pl.ds(c * N_PAD + off, 128)])


@functools.lru_cache(maxsize=None)
def _deg():
    return pl.kernel(
    _deg_body,
    out_type=jax.ShapeDtypeStruct((2 * N_PAD, 128), jnp.float32),
    mesh=_mesh(),
    scratch_types=[
        pltpu.VMEM_SHARED((N_PAD, 128), jnp.float32),
        pltpu.VMEM((8, 128), jnp.int32),
        pltpu.VMEM((128, 128), jnp.float32),
        pltpu.SemaphoreType.DMA,
    ],
    )


def _make_agg(W):
    def body(table, src2, dstb, zrow, out, acc, src_i, dst_i, rows, sem):
        c = lax.axis_index("c")
        s = lax.axis_index("s")
        pltpu.sync_copy(zrow, rows)
        for k in range(STRIPE // 128):
            pltpu.sync_copy(rows, acc.at[pl.ds(s * STRIPE + k * 128, 128)])
        plsc.subcore_barrier()

        rpt = IDX_ROWS // 16

        def step(g, carry):
            r0 = s * rpt + g * 8
            pltpu.sync_copy(src2.at[pl.ds(c * IDX_ROWS + r0, 8)], src_i)
            pltpu.sync_copy(dstb.at[pl.ds(r0, 8)], dst_i)
            for j in range(8):
                pltpu.async_copy(table.at[src_i.at[j]], rows, sem).wait()
                pltpu.sync_copy(rows, acc.at[dst_i.at[j]], add=True)
            return carry

        lax.fori_loop(0, rpt // 8, step, 0)
        plsc.subcore_barrier()
        for k in range(STRIPE // 128):
            off = s * STRIPE + k * 128
            pltpu.sync_copy(acc.at[pl.ds(off, 128)], rows)
            pltpu.sync_copy(rows, out.at[pl.ds(c * N_PAD + off, 128)])

    return pl.kernel(
        body,
        out_type=jax.ShapeDtypeStruct((2 * N_PAD, W), jnp.float32),
        mesh=_mesh(),
        scratch_types=[
            pltpu.VMEM_SHARED((N_PAD, W), jnp.float32),
            pltpu.VMEM((8, 128), jnp.int32),
            pltpu.VMEM((8, 128), jnp.int32),
            pltpu.VMEM((128, W), jnp.float32),
            pltpu.SemaphoreType.DMA,
        ],
    )


@functools.lru_cache(maxsize=None)
def _agg12():
    return _make_agg(128)


def _mm0_body(sums_ref, deg2_ref, x_ref, wl_ref, wr_ref, b_ref,
              h_ref, stats_ref, deg_ref):
    j = pl.program_id(0)
    ssum = sums_ref[0] + sums_ref[1]
    d2 = deg2_ref[0] + deg2_ref[1]
    deg_ref[...] = d2[:, :16]
    inv = 1.0 / jnp.maximum(d2[:, 0:1], 1.0)
    mean = ssum * inv
    h = (jnp.dot(mean, wl_ref[...], preferred_element_type=jnp.float32)
         + jnp.dot(x_ref[...], wr_ref[...], preferred_element_type=jnp.float32)
         + b_ref[...])
    h_ref[...] = h

    @pl.when(j == 0)
    def _():
        stats_ref[...] = jnp.zeros_like(stats_ref)

    stats_ref[0:1, :] = stats_ref[0:1, :] + jnp.sum(h, 0, keepdims=True)
    stats_ref[1:2, :] = stats_ref[1:2, :] + jnp.sum(h * h, 0, keepdims=True)


def _mm12_body(sums_ref, deg_ref, xin_ref, wl_ref, wr_ref, b_ref,
               h_ref, stats_ref):
    j = pl.program_id(0)
    ssum = jnp.concatenate([sums_ref[0], sums_ref[1]], axis=1)
    xcat = jnp.concatenate([xin_ref[0], xin_ref[1]], axis=1)
    inv = 1.0 / jnp.maximum(deg_ref[:, 0:1], 1.0)
    mean = ssum * inv
    h = (jnp.dot(mean, wl_ref[...], preferred_element_type=jnp.float32)
         + jnp.dot(xcat, wr_ref[...], preferred_element_type=jnp.float32)
         + b_ref[...])
    h_ref[...] = h

    @pl.when(j == 0)
    def _():
        stats_ref[...] = jnp.zeros_like(stats_ref)

    stats_ref[0:1, :] = stats_ref[0:1, :] + jnp.sum(h, 0, keepdims=True)
    stats_ref[1:2, :] = stats_ref[1:2, :] + jnp.sum(h * h, 0, keepdims=True)


def _bn_planes_body(h_ref, stats_ref, g_ref, b_ref, out_ref):
    st = stats_ref[...]
    mu = st[0:1, :] * (1.0 / N)
    var = st[1:2, :] * (1.0 / N) - mu * mu
    scale = g_ref[...] * lax.rsqrt(var + 1e-5)
    y = (h_ref[...] - mu) * scale + b_ref[...]
    y = jnp.maximum(y, 0.0)
    out_ref[0] = y[:, :128]
    out_ref[1] = y[:, 128:]


def _bn_res_planes_body(h_ref, stats_ref, g_ref, b_ref, xin_ref, out_ref):
    st = stats_ref[...]
    mu = st[0:1, :] * (1.0 / N)
    var = st[1:2, :] * (1.0 / N) - mu * mu
    scale = g_ref[...] * lax.rsqrt(var + 1e-5)
    y = (h_ref[...] - mu) * scale + b_ref[...]
    y = jnp.maximum(y, 0.0)
    out_ref[0] = xin_ref[0] + 0.3 * y[:, :128]
    out_ref[1] = xin_ref[1] + 0.3 * y[:, 128:]


def _bn_final_body(h_ref, stats_ref, g_ref, b_ref, out_ref):
    st = stats_ref[...]
    mu = st[0:1, :] * (1.0 / N)
    var = st[1:2, :] * (1.0 / N) - mu * mu
    scale = g_ref[...] * lax.rsqrt(var + 1e-5)
    y = (h_ref[...] - mu) * scale + b_ref[...]
    out_ref[...] = jnp.maximum(y, 0.0)


def _spec(shape, imap):
    return pl.BlockSpec(shape, imap)


def _mm0_call(sums, deg2, x, wl, wr, b):
    return pl.pallas_call(
        _mm0_body,
        grid=(GRID,),
        in_specs=[
            _spec((2, BN, 128), lambda j: (0, j, 0)),
            _spec((2, BN, 128), lambda j: (0, j, 0)),
            _spec((BN, D_IN), lambda j: (j, 0)),
            _spec((D_IN, H), lambda j: (0, 0)),
            _spec((D_IN, H), lambda j: (0, 0)),
            _spec((1, H), lambda j: (0, 0)),
        ],
        out_specs=[
            _spec((BN, H), lambda j: (j, 0)),
            _spec((8, H), lambda j: (0, 0)),
            _spec((BN, 16), lambda j: (j, 0)),
        ],
        out_shape=[
            jax.ShapeDtypeStruct((N, H), jnp.float32),
            jax.ShapeDtypeStruct((8, H), jnp.float32),
            jax.ShapeDtypeStruct((N_PAD, 16), jnp.float32),
        ],
    )(sums, deg2, x, wl, wr, b)


def _mm12_call(sums, deg, xin, wl, wr, b):
    return pl.pallas_call(
        _mm12_body,
        grid=(GRID,),
        in_specs=[
            _spec((2, BN, 128), lambda j: (0, j, 0)),
            _spec((BN, 16), lambda j: (j, 0)),
            _spec((2, BN, 128), lambda j: (0, j, 0)),
            _spec((H, H), lambda j: (0, 0)),
            _spec((H, H), lambda j: (0, 0)),
            _spec((1, H), lambda j: (0, 0)),
        ],
        out_specs=[
            _spec((BN, H), lambda j: (j, 0)),
            _spec((8, H), lambda j: (0, 0)),
        ],
        out_shape=[
            jax.ShapeDtypeStruct((N, H), jnp.float32),
            jax.ShapeDtypeStruct((8, H), jnp.float32),
        ],
    )(sums, deg, xin, wl, wr, b)


def _bn_planes_call(h, stats, g, b, xin=None):
    if xin is None:
        return pl.pallas_call(
            _bn_planes_body,
            grid=(GRID,),
            in_specs=[
                _spec((BN, H), lambda j: (j, 0)),
                _spec((8, H), lambda j: (0, 0)),
                _spec((1, H), lambda j: (0, 0)),
                _spec((1, H), lambda j: (0, 0)),
            ],
            out_specs=_spec((2, BN, 128), lambda j: (0, j, 0)),
            out_shape=jax.ShapeDtypeStruct((2, N_PAD, 128), jnp.float32),
        )(h, stats, g, b)
    return pl.pallas_call(
        _bn_res_planes_body,
        grid=(GRID,),
        in_specs=[
            _spec((BN, H), lambda j: (j, 0)),
            _spec((8, H), lambda j: (0, 0)),
            _spec((1, H), lambda j: (0, 0)),
            _spec((1, H), lambda j: (0, 0)),
            _spec((2, BN, 128), lambda j: (0, j, 0)),
        ],
        out_specs=_spec((2, BN, 128), lambda j: (0, j, 0)),
        out_shape=jax.ShapeDtypeStruct((2, N_PAD, 128), jnp.float32),
    )(h, stats, g, b, xin)


def _bn_final_call(h, stats, g, b):
    return pl.pallas_call(
        _bn_final_body,
        grid=(GRID,),
        in_specs=[
            _spec((BN, H), lambda j: (j, 0)),
            _spec((8, H), lambda j: (0, 0)),
            _spec((1, H), lambda j: (0, 0)),
            _spec((1, H), lambda j: (0, 0)),
        ],
        out_specs=_spec((BN, H), lambda j: (j, 0)),
        out_shape=jax.ShapeDtypeStruct((N, H), jnp.float32),
    )(h, stats, g, b)


def kernel(x, edge_index, W_l0, b_l0, W_r0, b_r0, gamma0, beta0,
           W_l1, b_l1, W_r1, b_r1, gamma1, beta1,
           W_l2, b_l2, W_r2, b_r2, gamma2, beta2):
    src = edge_index[0]
    dst = edge_index[1]
    pad = E_PAD - E
    srcb = jnp.concatenate([src, jnp.zeros((pad,), jnp.int32)]).reshape(IDX_ROWS, 128)
    dstb = jnp.concatenate([dst, jnp.full((pad,), N, jnp.int32)]).reshape(IDX_ROWS, 128)
    src2 = jnp.concatenate([srcb, srcb + N_PAD], axis=0)

    xpad = jnp.pad(x, ((0, N_PAD - N), (0, 0)))
    zrow128 = jnp.zeros((128, 128), jnp.float32)
    ones128 = jnp.ones((128, 128), jnp.float32)

    b0 = (b_l0 + b_r0).reshape(1, H)
    b1 = (b_l1 + b_r1).reshape(1, H)
    b2 = (b_l2 + b_r2).reshape(1, H)
    g0 = gamma0.reshape(1, H)
    g1 = gamma1.reshape(1, H)
    g2 = gamma2.reshape(1, H)
    t0 = beta0.reshape(1, H)
    t1 = beta1.reshape(1, H)
    t2 = beta2.reshape(1, H)

    sums0 = _agg0()(xpad, srcb, dstb, zrow128)
    deg2 = _deg()(dstb, zrow128, ones128)
    h0, st0, deg16 = _mm0_call(sums0.reshape(2, N_PAD, 128),
                               deg2.reshape(2, N_PAD, 128), x, W_l0, W_r0, b0)
    planes0 = _bn_planes_call(h0, st0, g0, t0)

    sums1 = _agg12()(planes0.reshape(2 * N_PAD, 128), src2, dstb, zrow128)
    h1, st1 = _mm12_call(sums1.reshape(2, N_PAD, 128), deg16, planes0,
                         W_l1, W_r1, b1)
    planes1 = _bn_planes_call(h1, st1, g1, t1, xin=planes0)

    sums2 = _agg12()(planes1.reshape(2 * N_PAD, 128), src2, dstb, zrow128)
    h2, st2 = _mm12_call(sums2.reshape(2, N_PAD, 128), deg16, planes1,
                         W_l2, W_r2, b2)
    return _bn_final_call(h2, st2, g2, t2)


# speedup vs baseline: 3.1765x; 1.0845x over previous
import jax
import jax.numpy as jnp
from jax import lax
from jax.experimental import pallas as pl
from jax.experimental.pallas import tpu as pltpu
from jax.experimental.pallas import tpu_sc as plsc

N = 10000
E = 320000
D_IN = 128
H = 256

N_PAD = 10240
E_PAD = 327680
IDX_ROWS = E_PAD // 128
STRIPE = N_PAD // 16
BN = 400
GRID = N // BN

import functools


@functools.lru_cache(maxsize=None)
def _mesh():
    return plsc.VectorSubcoreMesh(core_axis_name="c", subcore_axis_name="s")


_CHUNKS = 8


def _make_agg(plane_split):
    n_groups = (IDX_ROWS // 16 if plane_split else IDX_ROWS // 32) // _CHUNKS

    def body(table, srci, dstb, zrow, out,
             acc, src_i, dst_i, rows0, rows1, semG, semS):
        c = lax.axis_index("c")
        s = lax.axis_index("s")
        rows_ = (rows0, rows1)
        if plane_split:
            base = s * (IDX_ROWS // 16)
            src_off = c * IDX_ROWS
        else:
            base = (c * 16 + s) * (IDX_ROWS // 32)
            src_off = 0

        pltpu.sync_copy(zrow, rows0)
        for k in range(STRIPE // 128):
            pltpu.sync_copy(rows0, acc.at[pl.ds(s * STRIPE + k * 128, 128)])

        def load_idx(gidx):
            r0 = base + gidx * _CHUNKS
            pltpu.sync_copy(srci.at[pl.ds(src_off + r0, _CHUNKS)], src_i)
            pltpu.sync_copy(dstb.at[pl.ds(r0, _CHUNKS)], dst_i)

        load_idx(0)
        pltpu.async_copy(table.at[src_i.at[0]], rows0, semG)
        plsc.subcore_barrier()

        def grp(g, carry):
            for j in range(_CHUNKS):
                b = rows_[j % 2]
                nb = rows_[(j + 1) % 2]
                pltpu.make_async_copy(table.at[pl.ds(0, 128)], b, semG).wait()
                pltpu.async_copy(b, acc.at[dst_i.at[j]], semS, add=True)
                if j >= 1:
                    pltpu.make_async_copy(table.at[pl.ds(0, 128)], nb,
                                          semS).wait()
                if j < _CHUNKS - 1:
                    pltpu.async_copy(table.at[src_i.at[j + 1]], nb, semG)
            pltpu.make_async_copy(table.at[pl.ds(0, 128)],
                                  rows_[(_CHUNKS - 1) % 2], semS).wait()

            @pl.when(g < n_groups - 1)
            def _():
                load_idx(g + 1)
                pltpu.async_copy(table.at[src_i.at[0]], rows_[0], semG)

            return carry

        lax.fori_loop(0, n_groups, grp, 0)
        plsc.subcore_barrier()
        for k in range(STRIPE // 128):
            off = s * STRIPE + k * 128
            pltpu.sync_copy(acc.at[pl.ds(off, 128)], rows0)
            pltpu.sync_copy(rows0, out.at[pl.ds(c * N_PAD + off, 128)])

    return pl.kernel(
        body,
        out_type=jax.ShapeDtypeStruct((2 * N_PAD, 128), jnp.float32),
        mesh=_mesh(),
        scratch_types=[
            pltpu.VMEM_SHARED((N_PAD, 128), jnp.float32),
            pltpu.VMEM((_CHUNKS, 128), jnp.int32),
            pltpu.VMEM((_CHUNKS, 128), jnp.int32),
            pltpu.VMEM((128, 128), jnp.float32),
            pltpu.VMEM((128, 128), jnp.float32),
            pltpu.SemaphoreType.DMA,
            pltpu.SemaphoreType.DMA,
        ],
    )


@functools.lru_cache(maxsize=None)
def _agg0():
    return _make_agg(False)


def _deg_body(dstb, zrow, ones128, out, dacc, dst_i, buf, sem):
    c = lax.axis_index("c")
    s = lax.axis_index("s")
    w = c * 16 + s
    pltpu.sync_copy(zrow, buf)
    for k in range(STRIPE // 128):
        pltpu.sync_copy(buf, dacc.at[pl.ds(s * STRIPE + k * 128, 128)])
    pltpu.sync_copy(ones128, buf)
    plsc.subcore_barrier()

    rpw = IDX_ROWS // 32

    def step(g, carry):
        pltpu.sync_copy(dstb.at[pl.ds(w * rpw + g * 8, 8)], dst_i)
        for j in range(8):
            pltpu.sync_copy(buf, dacc.at[dst_i.at[j]], add=True)
        return carry

    lax.fori_loop(0, rpw // 8, step, 0)
    plsc.subcore_barrier()
    for k in range(STRIPE // 128):
        off = s * STRIPE + k * 128
        pltpu.sync_copy(dacc.at[pl.ds(off, 128)], buf)
        pltpu.sync_copy(buf, out.at[pl.ds(c * N_PAD + off, 128)])


@functools.lru_cache(maxsize=None)
def _deg():
    return pl.kernel(
    _deg_body,
    out_type=jax.ShapeDtypeStruct((2 * N_PAD, 128), jnp.float32),
    mesh=_mesh(),
    scratch_types=[
        pltpu.VMEM_SHARED((N_PAD, 128), jnp.float32),
        pltpu.VMEM((8, 128), jnp.int32),
        pltpu.VMEM((128, 128), jnp.float32),
        pltpu.SemaphoreType.DMA,
    ],
    )


@functools.lru_cache(maxsize=None)
def _agg12():
    return _make_agg(True)


def _mm0_body(sums_ref, deg2_ref, x_ref, wl_ref, wr_ref, b_ref,
              h_ref, stats_ref, deg_ref):
    j = pl.program_id(0)
    ssum = sums_ref[0] + sums_ref[1]
    d2 = deg2_ref[0] + deg2_ref[1]
    deg_ref[...] = d2[:, :16]
    inv = 1.0 / jnp.maximum(d2[:, 0:1], 1.0)
    mean = ssum * inv
    h = (jnp.dot(mean, wl_ref[...], preferred_element_type=jnp.float32)
         + jnp.dot(x_ref[...], wr_ref[...], preferred_element_type=jnp.float32)
         + b_ref[...])
    h_ref[...] = h

    @pl.when(j == 0)
    def _():
        stats_ref[...] = jnp.zeros_like(stats_ref)

    stats_ref[0:1, :] = stats_ref[0:1, :] + jnp.sum(h, 0, keepdims=True)
    stats_ref[1:2, :] = stats_ref[1:2, :] + jnp.sum(h * h, 0, keepdims=True)


def _mm12_body(sums_ref, deg_ref, xin_ref, wl_ref, wr_ref, b_ref,
               h_ref, stats_ref):
    j = pl.program_id(0)
    ssum = jnp.concatenate([sums_ref[0], sums_ref[1]], axis=1)
    xcat = jnp.concatenate([xin_ref[0], xin_ref[1]], axis=1)
    inv = 1.0 / jnp.maximum(deg_ref[:, 0:1], 1.0)
    mean = ssum * inv
    h = (jnp.dot(mean, wl_ref[...], preferred_element_type=jnp.float32)
         + jnp.dot(xcat, wr_ref[...], preferred_element_type=jnp.float32)
         + b_ref[...])
    h_ref[...] = h

    @pl.when(j == 0)
    def _():
        stats_ref[...] = jnp.zeros_like(stats_ref)

    stats_ref[0:1, :] = stats_ref[0:1, :] + jnp.sum(h, 0, keepdims=True)
    stats_ref[1:2, :] = stats_ref[1:2, :] + jnp.sum(h * h, 0, keepdims=True)


def _bn_planes_body(h_ref, stats_ref, g_ref, b_ref, out_ref):
    st = stats_ref[...]
    mu = st[0:1, :] * (1.0 / N)
    var = st[1:2, :] * (1.0 / N) - mu * mu
    scale = g_ref[...] * lax.rsqrt(var + 1e-5)
    y = (h_ref[...] - mu) * scale + b_ref[...]
    y = jnp.maximum(y, 0.0)
    out_ref[0] = y[:, :128]
    out_ref[1] = y[:, 128:]


def _bn_res_planes_body(h_ref, stats_ref, g_ref, b_ref, xin_ref, out_ref):
    st = stats_ref[...]
    mu = st[0:1, :] * (1.0 / N)
    var = st[1:2, :] * (1.0 / N) - mu * mu
    scale = g_ref[...] * lax.rsqrt(var + 1e-5)
    y = (h_ref[...] - mu) * scale + b_ref[...]
    y = jnp.maximum(y, 0.0)
    out_ref[0] = xin_ref[0] + 0.3 * y[:, :128]
    out_ref[1] = xin_ref[1] + 0.3 * y[:, 128:]


def _bn_final_body(h_ref, stats_ref, g_ref, b_ref, out_ref):
    st = stats_ref[...]
    mu = st[0:1, :] * (1.0 / N)
    var = st[1:2, :] * (1.0 / N) - mu * mu
    scale = g_ref[...] * lax.rsqrt(var + 1e-5)
    y = (h_ref[...] - mu) * scale + b_ref[...]
    out_ref[...] = jnp.maximum(y, 0.0)


def _spec(shape, imap):
    return pl.BlockSpec(shape, imap)


def _mm0_call(sums, deg2, x, wl, wr, b):
    return pl.pallas_call(
        _mm0_body,
        grid=(GRID,),
        in_specs=[
            _spec((2, BN, 128), lambda j: (0, j, 0)),
            _spec((2, BN, 128), lambda j: (0, j, 0)),
            _spec((BN, D_IN), lambda j: (j, 0)),
            _spec((D_IN, H), lambda j: (0, 0)),
            _spec((D_IN, H), lambda j: (0, 0)),
            _spec((1, H), lambda j: (0, 0)),
        ],
        out_specs=[
            _spec((BN, H), lambda j: (j, 0)),
            _spec((8, H), lambda j: (0, 0)),
            _spec((BN, 16), lambda j: (j, 0)),
        ],
        out_shape=[
            jax.ShapeDtypeStruct((N, H), jnp.float32),
            jax.ShapeDtypeStruct((8, H), jnp.float32),
            jax.ShapeDtypeStruct((N_PAD, 16), jnp.float32),
        ],
    )(sums, deg2, x, wl, wr, b)


def _mm12_call(sums, deg, xin, wl, wr, b):
    return pl.pallas_call(
        _mm12_body,
        grid=(GRID,),
        in_specs=[
            _spec((2, BN, 128), lambda j: (0, j, 0)),
            _spec((BN, 16), lambda j: (j, 0)),
            _spec((2, BN, 128), lambda j: (0, j, 0)),
            _spec((H, H), lambda j: (0, 0)),
            _spec((H, H), lambda j: (0, 0)),
            _spec((1, H), lambda j: (0, 0)),
        ],
        out_specs=[
            _spec((BN, H), lambda j: (j, 0)),
            _spec((8, H), lambda j: (0, 0)),
        ],
        out_shape=[
            jax.ShapeDtypeStruct((N, H), jnp.float32),
            jax.ShapeDtypeStruct((8, H), jnp.float32),
        ],
    )(sums, deg, xin, wl, wr, b)


def _bn_planes_call(h, stats, g, b, xin=None):
    if xin is None:
        return pl.pallas_call(
            _bn_planes_body,
            grid=(GRID,),
            in_specs=[
                _spec((BN, H), lambda j: (j, 0)),
                _spec((8, H), lambda j: (0, 0)),
                _spec((1, H), lambda j: (0, 0)),
                _spec((1, H), lambda j: (0, 0)),
            ],
            out_specs=_spec((2, BN, 128), lambda j: (0, j, 0)),
            out_shape=jax.ShapeDtypeStruct((2, N_PAD, 128), jnp.float32),
        )(h, stats, g, b)
    return pl.pallas_call(
        _bn_res_planes_body,
        grid=(GRID,),
        in_specs=[
            _spec((BN, H), lambda j: (j, 0)),
            _spec((8, H), lambda j: (0, 0)),
            _spec((1, H), lambda j: (0, 0)),
            _spec((1, H), lambda j: (0, 0)),
            _spec((2, BN, 128), lambda j: (0, j, 0)),
        ],
        out_specs=_spec((2, BN, 128), lambda j: (0, j, 0)),
        out_shape=jax.ShapeDtypeStruct((2, N_PAD, 128), jnp.float32),
    )(h, stats, g, b, xin)


def _bn_final_call(h, stats, g, b):
    return pl.pallas_call(
        _bn_final_body,
        grid=(GRID,),
        in_specs=[
            _spec((BN, H), lambda j: (j, 0)),
            _spec((8, H), lambda j: (0, 0)),
            _spec((1, H), lambda j: (0, 0)),
            _spec((1, H), lambda j: (0, 0)),
        ],
        out_specs=_spec((BN, H), lambda j: (j, 0)),
        out_shape=jax.ShapeDtypeStruct((N, H), jnp.float32),
    )(h, stats, g, b)


def kernel(x, edge_index, W_l0, b_l0, W_r0, b_r0, gamma0, beta0,
           W_l1, b_l1, W_r1, b_r1, gamma1, beta1,
           W_l2, b_l2, W_r2, b_r2, gamma2, beta2):
    src = edge_index[0]
    dst = edge_index[1]
    pad = E_PAD - E
    srcb = jnp.concatenate([src, jnp.zeros((pad,), jnp.int32)]).reshape(IDX_ROWS, 128)
    dstb = jnp.concatenate([dst, jnp.full((pad,), N, jnp.int32)]).reshape(IDX_ROWS, 128)
    src2 = jnp.concatenate([srcb, srcb + N_PAD], axis=0)

    xpad = jnp.pad(x, ((0, N_PAD - N), (0, 0)))
    zrow128 = jnp.zeros((128, 128), jnp.float32)
    ones128 = jnp.ones((128, 128), jnp.float32)

    b0 = (b_l0 + b_r0).reshape(1, H)
    b1 = (b_l1 + b_r1).reshape(1, H)
    b2 = (b_l2 + b_r2).reshape(1, H)
    g0 = gamma0.reshape(1, H)
    g1 = gamma1.reshape(1, H)
    g2 = gamma2.reshape(1, H)
    t0 = beta0.reshape(1, H)
    t1 = beta1.reshape(1, H)
    t2 = beta2.reshape(1, H)

    sums0 = _agg0()(xpad, srcb, dstb, zrow128)
    deg2 = _deg()(dstb, zrow128, ones128)
    h0, st0, deg16 = _mm0_call(sums0.reshape(2, N_PAD, 128),
                               deg2.reshape(2, N_PAD, 128), x, W_l0, W_r0, b0)
    planes0 = _bn_planes_call(h0, st0, g0, t0)

    sums1 = _agg12()(planes0.reshape(2 * N_PAD, 128), src2, dstb, zrow128)
    h1, st1 = _mm12_call(sums1.reshape(2, N_PAD, 128), deg16, planes0,
                         W_l1, W_r1, b1)
    planes1 = _bn_planes_call(h1, st1, g1, t1, xin=planes0)

    sums2 = _agg12()(planes1.reshape(2 * N_PAD, 128), src2, dstb, zrow128)
    h2, st2 = _mm12_call(sums2.reshape(2, N_PAD, 128), deg16, planes1,
                         W_l2, W_r2, b2)
    return _bn_final_call(h2, st2, g2, t2)


# speedup vs baseline: 3.2511x; 1.0235x over previous
import jax
import jax.numpy as jnp
from jax import lax
from jax.experimental import pallas as pl
from jax.experimental.pallas import tpu as pltpu
from jax.experimental.pallas import tpu_sc as plsc

N = 10000
E = 320000
D_IN = 128
H = 256

N_PAD = 10240
E_PAD = 327680
IDX_ROWS = E_PAD // 128
STRIPE = N_PAD // 16
BN = 400
GRID = N // BN

import functools


@functools.lru_cache(maxsize=None)
def _mesh():
    return plsc.VectorSubcoreMesh(core_axis_name="c", subcore_axis_name="s")


_CHUNKS = 16


def _make_agg(plane_split):
    n_groups = (IDX_ROWS // 16 if plane_split else IDX_ROWS // 32) // _CHUNKS

    def body(table, srci, dstb, zrow, out,
             acc, src_i, dst_i, rows0, rows1, semG, semS):
        c = lax.axis_index("c")
        s = lax.axis_index("s")
        rows_ = (rows0, rows1)
        if plane_split:
            base = s * (IDX_ROWS // 16)
            src_off = c * IDX_ROWS
        else:
            base = (c * 16 + s) * (IDX_ROWS // 32)
            src_off = 0

        pltpu.sync_copy(zrow, rows0)
        for k in range(STRIPE // 128):
            pltpu.sync_copy(rows0, acc.at[pl.ds(s * STRIPE + k * 128, 128)])

        def load_idx(gidx):
            r0 = base + gidx * _CHUNKS
            pltpu.sync_copy(srci.at[pl.ds(src_off + r0, _CHUNKS)], src_i)
            pltpu.sync_copy(dstb.at[pl.ds(r0, _CHUNKS)], dst_i)

        load_idx(0)
        pltpu.async_copy(table.at[src_i.at[0]], rows0, semG)
        plsc.subcore_barrier()

        def grp(g, carry):
            for j in range(_CHUNKS):
                b = rows_[j % 2]
                nb = rows_[(j + 1) % 2]
                pltpu.make_async_copy(table.at[pl.ds(0, 128)], b, semG).wait()
                pltpu.async_copy(b, acc.at[dst_i.at[j]], semS, add=True)
                if j >= 1:
                    pltpu.make_async_copy(table.at[pl.ds(0, 128)], nb,
                                          semS).wait()
                if j < _CHUNKS - 1:
                    pltpu.async_copy(table.at[src_i.at[j + 1]], nb, semG)
            pltpu.make_async_copy(table.at[pl.ds(0, 128)],
                                  rows_[(_CHUNKS - 1) % 2], semS).wait()

            @pl.when(g < n_groups - 1)
            def _():
                load_idx(g + 1)
                pltpu.async_copy(table.at[src_i.at[0]], rows_[0], semG)

            return carry

        lax.fori_loop(0, n_groups, grp, 0)
        plsc.subcore_barrier()
        for k in range(STRIPE // 128):
            off = s * STRIPE + k * 128
            pltpu.sync_copy(acc.at[pl.ds(off, 128)], rows0)
            pltpu.sync_copy(rows0, out.at[pl.ds(c * N_PAD + off, 128)])

    return pl.kernel(
        body,
        out_type=jax.ShapeDtypeStruct((2 * N_PAD, 128), jnp.float32),
        mesh=_mesh(),
        scratch_types=[
            pltpu.VMEM_SHARED((N_PAD, 128), jnp.float32),
            pltpu.VMEM((_CHUNKS, 128), jnp.int32),
            pltpu.VMEM((_CHUNKS, 128), jnp.int32),
            pltpu.VMEM((128, 128), jnp.float32),
            pltpu.VMEM((128, 128), jnp.float32),
            pltpu.SemaphoreType.DMA,
            pltpu.SemaphoreType.DMA,
        ],
    )


@functools.lru_cache(maxsize=None)
def _agg0():
    return _make_agg(False)


def _deg_body(dstb, zrow, ones128, out, dacc, dst_i, buf, sem):
    c = lax.axis_index("c")
    s = lax.axis_index("s")
    w = c * 16 + s
    pltpu.sync_copy(zrow, buf)
    for k in range(STRIPE // 128):
        pltpu.sync_copy(buf, dacc.at[pl.ds(s * STRIPE + k * 128, 128)])
    pltpu.sync_copy(ones128, buf)
    plsc.subcore_barrier()

    rpw = IDX_ROWS // 32

    def step(g, carry):
        pltpu.sync_copy(dstb.at[pl.ds(w * rpw + g * 8, 8)], dst_i)
        for j in range(8):
            pltpu.sync_copy(buf, dacc.at[dst_i.at[j]], add=True)
        return carry

    lax.fori_loop(0, rpw // 8, step, 0)
    plsc.subcore_barrier()
    for k in range(STRIPE // 128):
        off = s * STRIPE + k * 128
        pltpu.sync_copy(dacc.at[pl.ds(off, 128)], buf)
        pltpu.sync_copy(buf, out.at[pl.ds(c * N_PAD + off, 128)])


@functools.lru_cache(maxsize=None)
def _deg():
    return pl.kernel(
    _deg_body,
    out_type=jax.ShapeDtypeStruct((2 * N_PAD, 128), jnp.float32),
    mesh=_mesh(),
    scratch_types=[
        pltpu.VMEM_SHARED((N_PAD, 128), jnp.float32),
        pltpu.VMEM((8, 128), jnp.int32),
        pltpu.VMEM((128, 128), jnp.float32),
        pltpu.SemaphoreType.DMA,
    ],
    )


@functools.lru_cache(maxsize=None)
def _agg12():
    return _make_agg(True)


def _mm0_body(sums_ref, deg2_ref, x_ref, wl_ref, wr_ref, b_ref,
              h_ref, stats_ref, deg_ref):
    j = pl.program_id(0)
    ssum = sums_ref[0] + sums_ref[1]
    d2 = deg2_ref[0] + deg2_ref[1]
    deg_ref[...] = d2[:, :16]
    inv = 1.0 / jnp.maximum(d2[:, 0:1], 1.0)
    mean = ssum * inv
    h = (jnp.dot(mean, wl_ref[...], preferred_element_type=jnp.float32)
         + jnp.dot(x_ref[...], wr_ref[...], preferred_element_type=jnp.float32)
         + b_ref[...])
    h_ref[...] = h

    @pl.when(j == 0)
    def _():
        stats_ref[...] = jnp.zeros_like(stats_ref)

    stats_ref[0:1, :] = stats_ref[0:1, :] + jnp.sum(h, 0, keepdims=True)
    stats_ref[1:2, :] = stats_ref[1:2, :] + jnp.sum(h * h, 0, keepdims=True)


def _mm12_body(sums_ref, deg_ref, xin_ref, wl_ref, wr_ref, b_ref,
               h_ref, stats_ref):
    j = pl.program_id(0)
    ssum = jnp.concatenate([sums_ref[0], sums_ref[1]], axis=1)
    xcat = jnp.concatenate([xin_ref[0], xin_ref[1]], axis=1)
    inv = 1.0 / jnp.maximum(deg_ref[:, 0:1], 1.0)
    mean = ssum * inv
    h = (jnp.dot(mean, wl_ref[...], preferred_element_type=jnp.float32)
         + jnp.dot(xcat, wr_ref[...], preferred_element_type=jnp.float32)
         + b_ref[...])
    h_ref[...] = h

    @pl.when(j == 0)
    def _():
        stats_ref[...] = jnp.zeros_like(stats_ref)

    stats_ref[0:1, :] = stats_ref[0:1, :] + jnp.sum(h, 0, keepdims=True)
    stats_ref[1:2, :] = stats_ref[1:2, :] + jnp.sum(h * h, 0, keepdims=True)


def _bn_planes_body(h_ref, stats_ref, g_ref, b_ref, out_ref):
    st = stats_ref[...]
    mu = st[0:1, :] * (1.0 / N)
    var = st[1:2, :] * (1.0 / N) - mu * mu
    scale = g_ref[...] * lax.rsqrt(var + 1e-5)
    y = (h_ref[...] - mu) * scale + b_ref[...]
    y = jnp.maximum(y, 0.0)
    out_ref[0] = y[:, :128]
    out_ref[1] = y[:, 128:]


def _bn_res_planes_body(h_ref, stats_ref, g_ref, b_ref, xin_ref, out_ref):
    st = stats_ref[...]
    mu = st[0:1, :] * (1.0 / N)
    var = st[1:2, :] * (1.0 / N) - mu * mu
    scale = g_ref[...] * lax.rsqrt(var + 1e-5)
    y = (h_ref[...] - mu) * scale + b_ref[...]
    y = jnp.maximum(y, 0.0)
    out_ref[0] = xin_ref[0] + 0.3 * y[:, :128]
    out_ref[1] = xin_ref[1] + 0.3 * y[:, 128:]


def _bn_final_body(h_ref, stats_ref, g_ref, b_ref, out_ref):
    st = stats_ref[...]
    mu = st[0:1, :] * (1.0 / N)
    var = st[1:2, :] * (1.0 / N) - mu * mu
    scale = g_ref[...] * lax.rsqrt(var + 1e-5)
    y = (h_ref[...] - mu) * scale + b_ref[...]
    out_ref[...] = jnp.maximum(y, 0.0)


def _spec(shape, imap):
    return pl.BlockSpec(shape, imap)


def _mm0_call(sums, deg2, x, wl, wr, b):
    return pl.pallas_call(
        _mm0_body,
        grid=(GRID,),
        in_specs=[
            _spec((2, BN, 128), lambda j: (0, j, 0)),
            _spec((2, BN, 128), lambda j: (0, j, 0)),
            _spec((BN, D_IN), lambda j: (j, 0)),
            _spec((D_IN, H), lambda j: (0, 0)),
            _spec((D_IN, H), lambda j: (0, 0)),
            _spec((1, H), lambda j: (0, 0)),
        ],
        out_specs=[
            _spec((BN, H), lambda j: (j, 0)),
            _spec((8, H), lambda j: (0, 0)),
            _spec((BN, 16), lambda j: (j, 0)),
        ],
        out_shape=[
            jax.ShapeDtypeStruct((N, H), jnp.float32),
            jax.ShapeDtypeStruct((8, H), jnp.float32),
            jax.ShapeDtypeStruct((N_PAD, 16), jnp.float32),
        ],
    )(sums, deg2, x, wl, wr, b)


def _mm12_call(sums, deg, xin, wl, wr, b):
    return pl.pallas_call(
        _mm12_body,
        grid=(GRID,),
        in_specs=[
            _spec((2, BN, 128), lambda j: (0, j, 0)),
            _spec((BN, 16), lambda j: (j, 0)),
            _spec((2, BN, 128), lambda j: (0, j, 0)),
            _spec((H, H), lambda j: (0, 0)),
            _spec((H, H), lambda j: (0, 0)),
            _spec((1, H), lambda j: (0, 0)),
        ],
        out_specs=[
            _spec((BN, H), lambda j: (j, 0)),
            _spec((8, H), lambda j: (0, 0)),
        ],
        out_shape=[
            jax.ShapeDtypeStruct((N, H), jnp.float32),
            jax.ShapeDtypeStruct((8, H), jnp.float32),
        ],
    )(sums, deg, xin, wl, wr, b)


def _bn_planes_call(h, stats, g, b, xin=None):
    if xin is None:
        return pl.pallas_call(
            _bn_planes_body,
            grid=(GRID,),
            in_specs=[
                _spec((BN, H), lambda j: (j, 0)),
                _spec((8, H), lambda j: (0, 0)),
                _spec((1, H), lambda j: (0, 0)),
                _spec((1, H), lambda j: (0, 0)),
            ],
            out_specs=_spec((2, BN, 128), lambda j: (0, j, 0)),
            out_shape=jax.ShapeDtypeStruct((2, N_PAD, 128), jnp.float32),
        )(h, stats, g, b)
    return pl.pallas_call(
        _bn_res_planes_body,
        grid=(GRID,),
        in_specs=[
            _spec((BN, H), lambda j: (j, 0)),
            _spec((8, H), lambda j: (0, 0)),
            _spec((1, H), lambda j: (0, 0)),
            _spec((1, H), lambda j: (0, 0)),
            _spec((2, BN, 128), lambda j: (0, j, 0)),
        ],
        out_specs=_spec((2, BN, 128), lambda j: (0, j, 0)),
        out_shape=jax.ShapeDtypeStruct((2, N_PAD, 128), jnp.float32),
    )(h, stats, g, b, xin)


def _bn_final_call(h, stats, g, b):
    return pl.pallas_call(
        _bn_final_body,
        grid=(GRID,),
        in_specs=[
            _spec((BN, H), lambda j: (j, 0)),
            _spec((8, H), lambda j: (0, 0)),
            _spec((1, H), lambda j: (0, 0)),
            _spec((1, H), lambda j: (0, 0)),
        ],
        out_specs=_spec((BN, H), lambda j: (j, 0)),
        out_shape=jax.ShapeDtypeStruct((N, H), jnp.float32),
    )(h, stats, g, b)


def kernel(x, edge_index, W_l0, b_l0, W_r0, b_r0, gamma0, beta0,
           W_l1, b_l1, W_r1, b_r1, gamma1, beta1,
           W_l2, b_l2, W_r2, b_r2, gamma2, beta2):
    src = edge_index[0]
    dst = edge_index[1]
    pad = E_PAD - E
    srcb = jnp.concatenate([src, jnp.zeros((pad,), jnp.int32)]).reshape(IDX_ROWS, 128)
    dstb = jnp.concatenate([dst, jnp.full((pad,), N, jnp.int32)]).reshape(IDX_ROWS, 128)
    src2 = jnp.concatenate([srcb, srcb + N_PAD], axis=0)

    xpad = jnp.pad(x, ((0, N_PAD - N), (0, 0)))
    zrow128 = jnp.zeros((128, 128), jnp.float32)
    ones128 = jnp.ones((128, 128), jnp.float32)

    b0 = (b_l0 + b_r0).reshape(1, H)
    b1 = (b_l1 + b_r1).reshape(1, H)
    b2 = (b_l2 + b_r2).reshape(1, H)
    g0 = gamma0.reshape(1, H)
    g1 = gamma1.reshape(1, H)
    g2 = gamma2.reshape(1, H)
    t0 = beta0.reshape(1, H)
    t1 = beta1.reshape(1, H)
    t2 = beta2.reshape(1, H)

    sums0 = _agg0()(xpad, srcb, dstb, zrow128)
    deg2 = _deg()(dstb, zrow128, ones128)
    h0, st0, deg16 = _mm0_call(sums0.reshape(2, N_PAD, 128),
                               deg2.reshape(2, N_PAD, 128), x, W_l0, W_r0, b0)
    planes0 = _bn_planes_call(h0, st0, g0, t0)

    sums1 = _agg12()(planes0.reshape(2 * N_PAD, 128), src2, dstb, zrow128)
    h1, st1 = _mm12_call(sums1.reshape(2, N_PAD, 128), deg16, planes0,
                         W_l1, W_r1, b1)
    planes1 = _bn_planes_call(h1, st1, g1, t1, xin=planes0)

    sums2 = _agg12()(planes1.reshape(2 * N_PAD, 128), src2, dstb, zrow128)
    h2, st2 = _mm12_call(sums2.reshape(2, N_PAD, 128), deg16, planes1,
                         W_l2, W_r2, b2)
    return _bn_final_call(h2, st2, g2, t2)
